# Initial kernel scaffold; baseline (speedup 1.0000x reference)
import jax
import jax.numpy as jnp
from jax.experimental import pallas as pl


def kernel(x, edge_index, batch, emb_table, W1, b1, W2, b2, fc_W, fc_b):
    raise NotImplementedError("write your pallas kernel here")


# speedup vs baseline: 8.2540x; 8.2540x over previous
import functools

import jax
import jax.numpy as jnp
from jax import lax
from jax.experimental import pallas as pl
from jax.experimental.pallas import tpu as pltpu
from jax.experimental.pallas import tpu_sc as plsc

N_NODES = 10000
N_EDGES = 320000
NUM_EMB = 100000
D = 128
NUM_GRAPHS = 128

NC = 2
NS = 16
NW = NC * NS

NPAD = 10240
ROWS_W = NPAD // NW
ROWS_S = NPAD // NS

CH = 80
ECH_W = N_EDGES // NW
ENCH = ECH_W // CH
XNCH = ROWS_W // CH

BCH = 16
NBLK = 8
PNCH = BCH * NBLK
EPAD_W = PNCH * CH

@functools.lru_cache(maxsize=None)
def _sc_kernels():
    mesh = plsc.VectorSubcoreMesh(core_axis_name="c", subcore_axis_name="s",
                                  num_cores=NC, num_subcores=NS)

    @functools.partial(
        pl.kernel,
        out_type=(
            jax.ShapeDtypeStruct((NPAD, D), jnp.float32),
            jax.ShapeDtypeStruct((NC, NS, ROWS_S), jnp.float32),
        ),
        mesh=mesh,
        scratch_types=(
            pltpu.VMEM((XNCH, CH), jnp.int32),
            pltpu.VMEM((ENCH, CH), jnp.int32),
            pltpu.VMEM((CH, D), jnp.float32),
            pltpu.VMEM((CH,), jnp.float32),
            pltpu.VMEM((ROWS_S,), jnp.float32),
            pltpu.VMEM_SHARED((NPAD,), jnp.float32),
            pltpu.SemaphoreType.DMA,
        ),
    )
    def sc_gather_deg(emb_hbm, x_hbm, dst_hbm, ones_hbm, zeros_hbm,
                      h0_hbm, deg_hbm,
                      xi_v, dst_v, rows_v, ones_v, stage_v, deg_sh, sem):
        c = lax.axis_index("c")
        s = lax.axis_index("s")
        w = c * NS + s

        pltpu.sync_copy(ones_hbm, ones_v)
        pltpu.sync_copy(zeros_hbm, stage_v)
        pltpu.sync_copy(stage_v, deg_sh.at[pl.ds(s * ROWS_S, ROWS_S)])

        pltpu.sync_copy(x_hbm.at[w], xi_v)
        for j in range(XNCH):
            pltpu.async_copy(emb_hbm.at[xi_v.at[j]], rows_v, sem).wait()
            pltpu.sync_copy(rows_v, h0_hbm.at[pl.ds(w * ROWS_W + j * CH, CH)])

        plsc.subcore_barrier()

        pltpu.sync_copy(dst_hbm.at[w], dst_v)

        @pl.loop(0, ENCH)
        def _(j):
            pltpu.sync_copy(ones_v, deg_sh.at[dst_v.at[j]], add=True)

        plsc.subcore_barrier()

        pltpu.sync_copy(deg_sh.at[pl.ds(s * ROWS_S, ROWS_S)], stage_v)
        pltpu.sync_copy(stage_v, deg_hbm.at[c].at[s])

    @functools.partial(
        pl.kernel,
        out_type=jax.ShapeDtypeStruct((NC, NS, ROWS_S, D), jnp.float32),
        mesh=mesh,
        scratch_types=(
            pltpu.VMEM((BCH, CH), jnp.int32),
            pltpu.VMEM((BCH, CH), jnp.int32),
            pltpu.VMEM((CH, D), jnp.float32),
            pltpu.VMEM((CH, D), jnp.float32),
            pltpu.VMEM_SHARED((NPAD, D), jnp.float32),
            pltpu.SemaphoreType.DMA,
        ),
    )
    def sc_propagate(u_hbm, src_hbm, dst_hbm, zeros_hbm, acc_hbm,
                     src_v, dst_v, rows_v, stage_v, acc_sh, sem):
        c = lax.axis_index("c")
        s = lax.axis_index("s")
        w = c * NS + s

        pltpu.sync_copy(zeros_hbm, stage_v)

        @pl.loop(0, ROWS_S // CH)
        def _(k):
            pltpu.sync_copy(stage_v, acc_sh.at[pl.ds(s * ROWS_S + k * CH, CH)])

        plsc.subcore_barrier()

        for b in range(NBLK):
            pltpu.sync_copy(src_hbm.at[w].at[pl.ds(b * BCH, BCH)], src_v)
            pltpu.sync_copy(dst_hbm.at[w].at[pl.ds(b * BCH, BCH)], dst_v)

            @pl.loop(0, BCH)
            def _(j):
                pltpu.async_copy(u_hbm.at[src_v.at[j]], rows_v, sem).wait()
                pltpu.sync_copy(rows_v, acc_sh.at[dst_v.at[j]], add=True)

        plsc.subcore_barrier()

        @pl.loop(0, ROWS_S // CH)
        def _(k):
            pltpu.sync_copy(acc_sh.at[pl.ds(s * ROWS_S + k * CH, CH)], stage_v)
            pltpu.sync_copy(stage_v, acc_hbm.at[c].at[s].at[pl.ds(k * CH, CH)])

    return sc_gather_deg, sc_propagate


_BM = 2048


def _tc_first_body(h_ref, w_ref, da_ref, db_ref, u_ref, dinv_ref):
    dinv = lax.rsqrt(da_ref[...] + db_ref[...] + 1.0)
    dinv_ref[...] = dinv
    mm = jnp.dot(h_ref[...], w_ref[...], preferred_element_type=jnp.float32)
    u_ref[...] = mm * dinv


def _tc_first(h0, W1, dA, dB):
    grid = NPAD // _BM
    return pl.pallas_call(
        _tc_first_body,
        grid=(grid,),
        in_specs=[
            pl.BlockSpec((_BM, D), lambda i: (i, 0)),
            pl.BlockSpec((D, D), lambda i: (0, 0)),
            pl.BlockSpec((_BM, 1), lambda i: (i, 0)),
            pl.BlockSpec((_BM, 1), lambda i: (i, 0)),
        ],
        out_specs=[
            pl.BlockSpec((_BM, D), lambda i: (i, 0)),
            pl.BlockSpec((_BM, 1), lambda i: (i, 0)),
        ],
        out_shape=[
            jax.ShapeDtypeStruct((NPAD, D), jnp.float32),
            jax.ShapeDtypeStruct((NPAD, 1), jnp.float32),
        ],
    )(h0, W1, dA, dB)


def _tc_mid_body(a0_ref, a1_ref, up_ref, dinv_ref, b_ref, w_ref, u_ref):
    dinv = dinv_ref[...]
    h = (a0_ref[...] + a1_ref[...] + up_ref[...]) * dinv + b_ref[...]
    h = jnp.maximum(h, 0.0)
    u_ref[...] = jnp.dot(h, w_ref[...], preferred_element_type=jnp.float32) * dinv


def _tc_mid(a0, a1, u_prev, dinv, b, W):
    grid = NPAD // _BM
    return pl.pallas_call(
        _tc_mid_body,
        grid=(grid,),
        in_specs=[
            pl.BlockSpec((_BM, D), lambda i: (i, 0)),
            pl.BlockSpec((_BM, D), lambda i: (i, 0)),
            pl.BlockSpec((_BM, D), lambda i: (i, 0)),
            pl.BlockSpec((_BM, 1), lambda i: (i, 0)),
            pl.BlockSpec((1, D), lambda i: (0, 0)),
            pl.BlockSpec((D, D), lambda i: (0, 0)),
        ],
        out_specs=pl.BlockSpec((_BM, D), lambda i: (i, 0)),
        out_shape=jax.ShapeDtypeStruct((NPAD, D), jnp.float32),
    )(a0, a1, u_prev, dinv, b, W)


_BME = 1024


def _tc_pool_body(a0_ref, a1_ref, up_ref, dinv_ref, b_ref, batch_ref,
                  fcw_ref, fcb_ref, out_ref, sums_ref, counts_ref):
    i = pl.program_id(0)
    h = (a0_ref[...] + a1_ref[...] + up_ref[...]) * dinv_ref[...] + b_ref[...]
    h = jnp.maximum(h, 0.0)
    gids = lax.broadcasted_iota(jnp.int32, (NUM_GRAPHS, _BME), 0)
    sel = (gids == batch_ref[...]).astype(jnp.float32)

    @pl.when(i == 0)
    def _():
        sums_ref[...] = jnp.zeros_like(sums_ref)
        counts_ref[...] = jnp.zeros_like(counts_ref)

    sums_ref[...] += jnp.dot(sel, h, preferred_element_type=jnp.float32,
                             precision=lax.Precision.HIGHEST)
    counts_ref[...] += jnp.sum(sel, axis=1, keepdims=True)

    @pl.when(i == pl.num_programs(0) - 1)
    def _():
        pooled = sums_ref[...] / jnp.maximum(counts_ref[...], 1.0)
        out_ref[...] = (
            jnp.dot(pooled, fcw_ref[...], preferred_element_type=jnp.float32)
            + fcb_ref[...])


def _tc_pool(a0, a1, u_prev, dinv, b, batch2d, fc_W, fc_b2d):
    grid = NPAD // _BME
    return pl.pallas_call(
        _tc_pool_body,
        grid=(grid,),
        in_specs=[
            pl.BlockSpec((_BME, D), lambda i: (i, 0)),
            pl.BlockSpec((_BME, D), lambda i: (i, 0)),
            pl.BlockSpec((_BME, D), lambda i: (i, 0)),
            pl.BlockSpec((_BME, 1), lambda i: (i, 0)),
            pl.BlockSpec((1, D), lambda i: (0, 0)),
            pl.BlockSpec((1, _BME), lambda i: (0, i)),
            pl.BlockSpec((D, 1), lambda i: (0, 0)),
            pl.BlockSpec((1, 1), lambda i: (0, 0)),
        ],
        out_specs=pl.BlockSpec((NUM_GRAPHS, 1), lambda i: (0, 0)),
        out_shape=jax.ShapeDtypeStruct((NUM_GRAPHS, 1), jnp.float32),
        scratch_shapes=[
            pltpu.VMEM((NUM_GRAPHS, D), jnp.float32),
            pltpu.VMEM((NUM_GRAPHS, 1), jnp.float32),
        ],
    )(a0, a1, u_prev, dinv, b, batch2d, fc_W, fc_b2d)


def kernel(x, edge_index, batch, emb_table, W1, b1, W2, b2, fc_W, fc_b):
    x_pad = jnp.concatenate([x, jnp.zeros((NPAD - N_NODES,), jnp.int32)])
    x_r = x_pad.reshape(NW, XNCH, CH)
    dst_r = edge_index[1].reshape(NW, ENCH, CH)
    npad_e = EPAD_W - ECH_W
    src_p = jnp.concatenate(
        [edge_index[0].reshape(NW, ECH_W),
         jnp.zeros((NW, npad_e), jnp.int32)], axis=1).reshape(NW, PNCH, CH)
    dummy_dst = jnp.broadcast_to(
        N_NODES + jnp.arange(npad_e, dtype=jnp.int32) % (NPAD - N_NODES),
        (NW, npad_e))
    dst_p = jnp.concatenate(
        [edge_index[1].reshape(NW, ECH_W), dummy_dst],
        axis=1).reshape(NW, PNCH, CH)
    batch_pad = jnp.concatenate(
        [batch, jnp.full((NPAD - N_NODES,), NUM_GRAPHS, jnp.int32)])
    batch2d = batch_pad.reshape(1, NPAD)

    ones_ch = jnp.ones((CH,), jnp.float32)
    zeros_rs = jnp.zeros((ROWS_S,), jnp.float32)
    zeros_ch_d = jnp.zeros((CH, D), jnp.float32)

    sc_gather_deg, sc_propagate = _sc_kernels()
    h0, deg = sc_gather_deg(emb_table, x_r, dst_r, ones_ch, zeros_rs)
    dA = deg[0].reshape(NPAD, 1)
    dB = deg[1].reshape(NPAD, 1)

    u1, dinv = _tc_first(h0, W1, dA, dB)

    acc1 = sc_propagate(u1, src_p, dst_p, zeros_ch_d)
    a10 = acc1[0].reshape(NPAD, D)
    a11 = acc1[1].reshape(NPAD, D)

    u2 = _tc_mid(a10, a11, u1, dinv, b1.reshape(1, D), W2)

    acc2 = sc_propagate(u2, src_p, dst_p, zeros_ch_d)
    a20 = acc2[0].reshape(NPAD, D)
    a21 = acc2[1].reshape(NPAD, D)

    out = _tc_pool(a20, a21, u2, dinv, b2.reshape(1, D), batch2d,
                   fc_W, fc_b.reshape(1, 1))
    return out.reshape(NUM_GRAPHS)


# speedup vs baseline: 9.2318x; 1.1185x over previous
import functools

import jax
import jax.numpy as jnp
from jax import lax
from jax.experimental import pallas as pl
from jax.experimental.pallas import tpu as pltpu
from jax.experimental.pallas import tpu_sc as plsc

N_NODES = 10000
N_EDGES = 320000
NUM_EMB = 100000
D = 128
NUM_GRAPHS = 128

NC = 2
NS = 16
NW = NC * NS

NPAD = 10240
ROWS_W = NPAD // NW
ROWS_S = NPAD // NS

CH = 80
ECH_W = N_EDGES // NW
ENCH = ECH_W // CH
XNCH = ROWS_W // CH

BCH = 16
NBLK = 8
PNCH = BCH * NBLK
EPAD_W = PNCH * CH

@functools.lru_cache(maxsize=None)
def _sc_kernels():
    mesh = plsc.VectorSubcoreMesh(core_axis_name="c", subcore_axis_name="s",
                                  num_cores=NC, num_subcores=NS)

    @functools.partial(
        pl.kernel,
        out_type=(
            jax.ShapeDtypeStruct((NPAD, D), jnp.float32),
            jax.ShapeDtypeStruct((NC, NS, ROWS_S), jnp.float32),
        ),
        mesh=mesh,
        scratch_types=(
            pltpu.VMEM((XNCH, CH), jnp.int32),
            pltpu.VMEM((ENCH, CH), jnp.int32),
            pltpu.VMEM((CH, D), jnp.float32),
            pltpu.VMEM((CH,), jnp.float32),
            pltpu.VMEM((ROWS_S,), jnp.float32),
            pltpu.VMEM_SHARED((NPAD,), jnp.float32),
            pltpu.SemaphoreType.DMA,
        ),
    )
    def sc_gather_deg(emb_hbm, x_hbm, dst_hbm, ones_hbm, zeros_hbm,
                      h0_hbm, deg_hbm,
                      xi_v, dst_v, rows_v, ones_v, stage_v, deg_sh, sem):
        c = lax.axis_index("c")
        s = lax.axis_index("s")
        w = c * NS + s

        pltpu.sync_copy(ones_hbm, ones_v)
        pltpu.sync_copy(zeros_hbm, stage_v)
        pltpu.sync_copy(stage_v, deg_sh.at[pl.ds(s * ROWS_S, ROWS_S)])

        pltpu.sync_copy(x_hbm.at[w], xi_v)
        for j in range(XNCH):
            pltpu.async_copy(emb_hbm.at[xi_v.at[j]], rows_v, sem).wait()
            pltpu.sync_copy(rows_v, h0_hbm.at[pl.ds(w * ROWS_W + j * CH, CH)])

        plsc.subcore_barrier()

        pltpu.sync_copy(dst_hbm.at[w], dst_v)

        @pl.loop(0, ENCH)
        def _(j):
            pltpu.sync_copy(ones_v, deg_sh.at[dst_v.at[j]], add=True)

        plsc.subcore_barrier()

        pltpu.sync_copy(deg_sh.at[pl.ds(s * ROWS_S, ROWS_S)], stage_v)
        pltpu.sync_copy(stage_v, deg_hbm.at[c].at[s])

    @functools.partial(
        pl.kernel,
        out_type=jax.ShapeDtypeStruct((NC, NS, ROWS_S, D), jnp.float32),
        mesh=mesh,
        scratch_types=(
            pltpu.VMEM((3, BCH, CH), jnp.int32),
            pltpu.VMEM((3, BCH, CH), jnp.int32),
            pltpu.VMEM((2, CH, D), jnp.float32),
            pltpu.VMEM_SHARED((NPAD, D), jnp.float32),
            pltpu.SemaphoreType.DMA,
            pltpu.SemaphoreType.DMA,
            pltpu.SemaphoreType.DMA,
        ),
    )
    def sc_propagate(u_hbm, src_hbm, dst_hbm, zeros_hbm, acc_hbm,
                     src_v, dst_v, rows_v, acc_sh, gsem, ssem, isem):
        c = lax.axis_index("c")
        s = lax.axis_index("s")
        w = c * NS + s
        nrb = ROWS_S // CH

        def fire_gather(q, j, p):
            pltpu.async_copy(u_hbm.at[src_v.at[q].at[j]], rows_v.at[p], gsem)

        def wait_gather(p):
            pltpu.make_async_copy(u_hbm.at[pl.ds(0, CH)], rows_v.at[p],
                                  gsem).wait()

        def fire_scatter(q, j, p):
            pltpu.async_copy(rows_v.at[p], acc_sh.at[dst_v.at[q].at[j]],
                             ssem, add=True)

        def wait_ssem():
            pltpu.make_async_copy(rows_v.at[0], acc_sh.at[pl.ds(0, CH)],
                                  ssem).wait()

        def fire_idx(b):
            pltpu.async_copy(src_hbm.at[w].at[pl.ds(b * BCH, BCH)],
                             src_v.at[b % 3], isem)
            pltpu.async_copy(dst_hbm.at[w].at[pl.ds(b * BCH, BCH)],
                             dst_v.at[b % 3], isem)

        def wait_idx():
            for ref in (src_v, dst_v):
                pltpu.make_async_copy(src_hbm.at[0].at[pl.ds(0, BCH)],
                                      ref.at[0], isem).wait()

        pltpu.sync_copy(zeros_hbm, rows_v.at[0])
        for k in range(nrb):
            pltpu.async_copy(rows_v.at[0],
                             acc_sh.at[pl.ds(s * ROWS_S + k * CH, CH)], ssem)
        for k in range(nrb):
            wait_ssem()
        plsc.subcore_barrier()

        fire_idx(0)
        wait_idx()
        if NBLK > 1:
            fire_idx(1)
        fire_gather(0, 0, 0)
        wait_gather(0)
        fire_gather(0, 1, 1)
        fire_scatter(0, 0, 0)

        for b in range(NBLK):
            q = b % 3

            @pl.loop(1 if b == 0 else 0, BCH - 1)
            def _(j):
                g = b * BCH + j
                p = lax.rem(g, 2)
                wait_gather(p)
                wait_ssem()
                fire_gather(q, j + 1, 1 - p)
                fire_scatter(q, j, p)

            wait_gather(1)
            wait_ssem()
            if b + 1 < NBLK:
                wait_idx()
                if b + 2 < NBLK:
                    fire_idx(b + 2)
                fire_gather((b + 1) % 3, 0, 0)
            fire_scatter(q, BCH - 1, 1)

        wait_ssem()
        plsc.subcore_barrier()

        def fire_rd(k, p):
            pltpu.async_copy(acc_sh.at[pl.ds(s * ROWS_S + k * CH, CH)],
                             rows_v.at[p], gsem)

        def fire_wr(k, p):
            pltpu.async_copy(rows_v.at[p],
                             acc_hbm.at[c].at[s].at[pl.ds(k * CH, CH)], ssem)

        def wait_rd(p):
            pltpu.make_async_copy(u_hbm.at[pl.ds(0, CH)], rows_v.at[p],
                                  gsem).wait()

        def wait_wr():
            pltpu.make_async_copy(rows_v.at[0],
                                  acc_hbm.at[0].at[0].at[pl.ds(0, CH)],
                                  ssem).wait()

        fire_rd(0, 0)
        for k in range(nrb):
            p = k % 2
            wait_rd(p)
            if k >= 1:
                wait_wr()
            if k + 1 < nrb:
                fire_rd(k + 1, 1 - p)
            fire_wr(k, p)
        wait_wr()

    return sc_gather_deg, sc_propagate


_BM = 2048


def _tc_first_body(h_ref, w_ref, da_ref, db_ref, u_ref, dinv_ref):
    dinv = lax.rsqrt(da_ref[...] + db_ref[...] + 1.0)
    dinv_ref[...] = dinv
    mm = jnp.dot(h_ref[...], w_ref[...], preferred_element_type=jnp.float32)
    u_ref[...] = mm * dinv


def _tc_first(h0, W1, dA, dB):
    grid = NPAD // _BM
    return pl.pallas_call(
        _tc_first_body,
        grid=(grid,),
        in_specs=[
            pl.BlockSpec((_BM, D), lambda i: (i, 0)),
            pl.BlockSpec((D, D), lambda i: (0, 0)),
            pl.BlockSpec((_BM, 1), lambda i: (i, 0)),
            pl.BlockSpec((_BM, 1), lambda i: (i, 0)),
        ],
        out_specs=[
            pl.BlockSpec((_BM, D), lambda i: (i, 0)),
            pl.BlockSpec((_BM, 1), lambda i: (i, 0)),
        ],
        out_shape=[
            jax.ShapeDtypeStruct((NPAD, D), jnp.float32),
            jax.ShapeDtypeStruct((NPAD, 1), jnp.float32),
        ],
    )(h0, W1, dA, dB)


def _tc_mid_body(a0_ref, a1_ref, up_ref, dinv_ref, b_ref, w_ref, u_ref):
    dinv = dinv_ref[...]
    h = (a0_ref[...] + a1_ref[...] + up_ref[...]) * dinv + b_ref[...]
    h = jnp.maximum(h, 0.0)
    u_ref[...] = jnp.dot(h, w_ref[...], preferred_element_type=jnp.float32) * dinv


def _tc_mid(a0, a1, u_prev, dinv, b, W):
    grid = NPAD // _BM
    return pl.pallas_call(
        _tc_mid_body,
        grid=(grid,),
        in_specs=[
            pl.BlockSpec((_BM, D), lambda i: (i, 0)),
            pl.BlockSpec((_BM, D), lambda i: (i, 0)),
            pl.BlockSpec((_BM, D), lambda i: (i, 0)),
            pl.BlockSpec((_BM, 1), lambda i: (i, 0)),
            pl.BlockSpec((1, D), lambda i: (0, 0)),
            pl.BlockSpec((D, D), lambda i: (0, 0)),
        ],
        out_specs=pl.BlockSpec((_BM, D), lambda i: (i, 0)),
        out_shape=jax.ShapeDtypeStruct((NPAD, D), jnp.float32),
    )(a0, a1, u_prev, dinv, b, W)


_BME = 1024


def _tc_pool_body(a0_ref, a1_ref, up_ref, dinv_ref, b_ref, batch_ref,
                  fcw_ref, fcb_ref, out_ref, sums_ref, counts_ref):
    i = pl.program_id(0)
    h = (a0_ref[...] + a1_ref[...] + up_ref[...]) * dinv_ref[...] + b_ref[...]
    h = jnp.maximum(h, 0.0)
    gids = lax.broadcasted_iota(jnp.int32, (NUM_GRAPHS, _BME), 0)
    sel = (gids == batch_ref[...]).astype(jnp.float32)

    @pl.when(i == 0)
    def _():
        sums_ref[...] = jnp.zeros_like(sums_ref)
        counts_ref[...] = jnp.zeros_like(counts_ref)

    sums_ref[...] += jnp.dot(sel, h, preferred_element_type=jnp.float32,
                             precision=lax.Precision.HIGHEST)
    counts_ref[...] += jnp.sum(sel, axis=1, keepdims=True)

    @pl.when(i == pl.num_programs(0) - 1)
    def _():
        pooled = sums_ref[...] / jnp.maximum(counts_ref[...], 1.0)
        out_ref[...] = (
            jnp.dot(pooled, fcw_ref[...], preferred_element_type=jnp.float32)
            + fcb_ref[...])


def _tc_pool(a0, a1, u_prev, dinv, b, batch2d, fc_W, fc_b2d):
    grid = NPAD // _BME
    return pl.pallas_call(
        _tc_pool_body,
        grid=(grid,),
        in_specs=[
            pl.BlockSpec((_BME, D), lambda i: (i, 0)),
            pl.BlockSpec((_BME, D), lambda i: (i, 0)),
            pl.BlockSpec((_BME, D), lambda i: (i, 0)),
            pl.BlockSpec((_BME, 1), lambda i: (i, 0)),
            pl.BlockSpec((1, D), lambda i: (0, 0)),
            pl.BlockSpec((1, _BME), lambda i: (0, i)),
            pl.BlockSpec((D, 1), lambda i: (0, 0)),
            pl.BlockSpec((1, 1), lambda i: (0, 0)),
        ],
        out_specs=pl.BlockSpec((NUM_GRAPHS, 1), lambda i: (0, 0)),
        out_shape=jax.ShapeDtypeStruct((NUM_GRAPHS, 1), jnp.float32),
        scratch_shapes=[
            pltpu.VMEM((NUM_GRAPHS, D), jnp.float32),
            pltpu.VMEM((NUM_GRAPHS, 1), jnp.float32),
        ],
    )(a0, a1, u_prev, dinv, b, batch2d, fc_W, fc_b2d)


def kernel(x, edge_index, batch, emb_table, W1, b1, W2, b2, fc_W, fc_b):
    x_pad = jnp.concatenate([x, jnp.zeros((NPAD - N_NODES,), jnp.int32)])
    x_r = x_pad.reshape(NW, XNCH, CH)
    dst_r = edge_index[1].reshape(NW, ENCH, CH)
    npad_e = EPAD_W - ECH_W
    src_p = jnp.concatenate(
        [edge_index[0].reshape(NW, ECH_W),
         jnp.zeros((NW, npad_e), jnp.int32)], axis=1).reshape(NW, PNCH, CH)
    dummy_dst = jnp.broadcast_to(
        N_NODES + jnp.arange(npad_e, dtype=jnp.int32) % (NPAD - N_NODES),
        (NW, npad_e))
    dst_p = jnp.concatenate(
        [edge_index[1].reshape(NW, ECH_W), dummy_dst],
        axis=1).reshape(NW, PNCH, CH)
    batch_pad = jnp.concatenate(
        [batch, jnp.full((NPAD - N_NODES,), NUM_GRAPHS, jnp.int32)])
    batch2d = batch_pad.reshape(1, NPAD)

    ones_ch = jnp.ones((CH,), jnp.float32)
    zeros_rs = jnp.zeros((ROWS_S,), jnp.float32)
    zeros_ch_d = jnp.zeros((CH, D), jnp.float32)

    sc_gather_deg, sc_propagate = _sc_kernels()
    h0, deg = sc_gather_deg(emb_table, x_r, dst_r, ones_ch, zeros_rs)
    dA = deg[0].reshape(NPAD, 1)
    dB = deg[1].reshape(NPAD, 1)

    u1, dinv = _tc_first(h0, W1, dA, dB)

    acc1 = sc_propagate(u1, src_p, dst_p, zeros_ch_d)
    a10 = acc1[0].reshape(NPAD, D)
    a11 = acc1[1].reshape(NPAD, D)

    u2 = _tc_mid(a10, a11, u1, dinv, b1.reshape(1, D), W2)

    acc2 = sc_propagate(u2, src_p, dst_p, zeros_ch_d)
    a20 = acc2[0].reshape(NPAD, D)
    a21 = acc2[1].reshape(NPAD, D)

    out = _tc_pool(a20, a21, u2, dinv, b2.reshape(1, D), batch2d,
                   fc_W, fc_b.reshape(1, 1))
    return out.reshape(NUM_GRAPHS)


# speedup vs baseline: 9.6090x; 1.0409x over previous
import functools

import jax
import jax.numpy as jnp
from jax import lax
from jax.experimental import pallas as pl
from jax.experimental.pallas import tpu as pltpu
from jax.experimental.pallas import tpu_sc as plsc

N_NODES = 10000
N_EDGES = 320000
NUM_EMB = 100000
D = 128
NUM_GRAPHS = 128

NC = 2
NS = 16
NW = NC * NS

NPAD = 10240
ROWS_W = NPAD // NW
ROWS_S = NPAD // NS

CH = 80
ECH_W = N_EDGES // NW
ENCH = ECH_W // CH
XNCH = ROWS_W // CH

PCH = 128
BCH = 4
NBLK = 20
PNCH = BCH * NBLK
EPAD_W = PNCH * PCH

@functools.lru_cache(maxsize=None)
def _sc_kernels():
    mesh = plsc.VectorSubcoreMesh(core_axis_name="c", subcore_axis_name="s",
                                  num_cores=NC, num_subcores=NS)

    @functools.partial(
        pl.kernel,
        out_type=(
            jax.ShapeDtypeStruct((NPAD, D), jnp.float32),
            jax.ShapeDtypeStruct((NC, NS, ROWS_S), jnp.float32),
        ),
        mesh=mesh,
        scratch_types=(
            pltpu.VMEM((XNCH, CH), jnp.int32),
            pltpu.VMEM((ENCH, CH), jnp.int32),
            pltpu.VMEM((CH, D), jnp.float32),
            pltpu.VMEM((CH,), jnp.float32),
            pltpu.VMEM((ROWS_S,), jnp.float32),
            pltpu.VMEM_SHARED((NPAD,), jnp.float32),
            pltpu.SemaphoreType.DMA,
        ),
    )
    def sc_gather_deg(emb_hbm, x_hbm, dst_hbm, ones_hbm, zeros_hbm,
                      h0_hbm, deg_hbm,
                      xi_v, dst_v, rows_v, ones_v, stage_v, deg_sh, sem):
        c = lax.axis_index("c")
        s = lax.axis_index("s")
        w = c * NS + s

        pltpu.sync_copy(ones_hbm, ones_v)
        pltpu.sync_copy(zeros_hbm, stage_v)
        pltpu.sync_copy(stage_v, deg_sh.at[pl.ds(s * ROWS_S, ROWS_S)])

        pltpu.sync_copy(x_hbm.at[w], xi_v)
        for j in range(XNCH):
            pltpu.async_copy(emb_hbm.at[xi_v.at[j]], rows_v, sem).wait()
            pltpu.sync_copy(rows_v, h0_hbm.at[pl.ds(w * ROWS_W + j * CH, CH)])

        plsc.subcore_barrier()

        pltpu.sync_copy(dst_hbm.at[w], dst_v)

        @pl.loop(0, ENCH)
        def _(j):
            pltpu.sync_copy(ones_v, deg_sh.at[dst_v.at[j]], add=True)

        plsc.subcore_barrier()

        pltpu.sync_copy(deg_sh.at[pl.ds(s * ROWS_S, ROWS_S)], stage_v)
        pltpu.sync_copy(stage_v, deg_hbm.at[c].at[s])

    @functools.partial(
        pl.kernel,
        out_type=jax.ShapeDtypeStruct((NC, NS, ROWS_S, D), jnp.float32),
        mesh=mesh,
        scratch_types=(
            pltpu.VMEM((3, BCH, PCH), jnp.int32),
            pltpu.VMEM((3, BCH, PCH), jnp.int32),
            pltpu.VMEM((2, PCH, D), jnp.float32),
            pltpu.VMEM_SHARED((NPAD, D), jnp.float32),
            pltpu.SemaphoreType.DMA,
            pltpu.SemaphoreType.DMA,
            pltpu.SemaphoreType.DMA,
        ),
    )
    def sc_propagate(u_hbm, src_hbm, dst_hbm, zeros_hbm, acc_hbm,
                     src_v, dst_v, rows_v, acc_sh, gsem, ssem, isem):
        c = lax.axis_index("c")
        s = lax.axis_index("s")
        w = c * NS + s
        nrb = ROWS_S // PCH

        def fire_gather(q, j, p):
            pltpu.async_copy(u_hbm.at[src_v.at[q].at[j]], rows_v.at[p], gsem)

        def wait_gather(p):
            pltpu.make_async_copy(u_hbm.at[pl.ds(0, PCH)], rows_v.at[p],
                                  gsem).wait()

        def fire_scatter(q, j, p):
            pltpu.async_copy(rows_v.at[p], acc_sh.at[dst_v.at[q].at[j]],
                             ssem, add=True)

        def wait_ssem():
            pltpu.make_async_copy(rows_v.at[0], acc_sh.at[pl.ds(0, PCH)],
                                  ssem).wait()

        def fire_idx(b):
            pltpu.async_copy(src_hbm.at[w].at[pl.ds(b * BCH, BCH)],
                             src_v.at[b % 3], isem)
            pltpu.async_copy(dst_hbm.at[w].at[pl.ds(b * BCH, BCH)],
                             dst_v.at[b % 3], isem)

        def wait_idx():
            for ref in (src_v, dst_v):
                pltpu.make_async_copy(src_hbm.at[0].at[pl.ds(0, BCH)],
                                      ref.at[0], isem).wait()

        pltpu.sync_copy(zeros_hbm, rows_v.at[0])
        for k in range(nrb):
            pltpu.async_copy(rows_v.at[0],
                             acc_sh.at[pl.ds(s * ROWS_S + k * PCH, PCH)], ssem)
        for k in range(nrb):
            wait_ssem()
        plsc.subcore_barrier()

        fire_idx(0)
        wait_idx()
        if NBLK > 1:
            fire_idx(1)
        fire_gather(0, 0, 0)
        wait_gather(0)
        fire_gather(0, 1, 1)
        fire_scatter(0, 0, 0)

        for b in range(NBLK):
            q = b % 3

            @pl.loop(1 if b == 0 else 0, BCH - 1)
            def _(j):
                g = b * BCH + j
                p = lax.rem(g, 2)
                wait_gather(p)
                wait_ssem()
                fire_gather(q, j + 1, 1 - p)
                fire_scatter(q, j, p)

            wait_gather(1)
            wait_ssem()
            if b + 1 < NBLK:
                wait_idx()
                if b + 2 < NBLK:
                    fire_idx(b + 2)
                fire_gather((b + 1) % 3, 0, 0)
            fire_scatter(q, BCH - 1, 1)

        wait_ssem()
        plsc.subcore_barrier()

        def fire_rd(k, p):
            pltpu.async_copy(acc_sh.at[pl.ds(s * ROWS_S + k * PCH, PCH)],
                             rows_v.at[p], gsem)

        def fire_wr(k, p):
            pltpu.async_copy(rows_v.at[p],
                             acc_hbm.at[c].at[s].at[pl.ds(k * PCH, PCH)], ssem)

        def wait_rd(p):
            pltpu.make_async_copy(u_hbm.at[pl.ds(0, PCH)], rows_v.at[p],
                                  gsem).wait()

        def wait_wr():
            pltpu.make_async_copy(rows_v.at[0],
                                  acc_hbm.at[0].at[0].at[pl.ds(0, PCH)],
                                  ssem).wait()

        fire_rd(0, 0)
        for k in range(nrb):
            p = k % 2
            wait_rd(p)
            if k >= 1:
                wait_wr()
            if k + 1 < nrb:
                fire_rd(k + 1, 1 - p)
            fire_wr(k, p)
        wait_wr()

    return sc_gather_deg, sc_propagate


_BM = 2048


def _tc_first_body(h_ref, w_ref, da_ref, db_ref, u_ref, dinv_ref):
    dinv = lax.rsqrt(da_ref[...] + db_ref[...] + 1.0)
    dinv_ref[...] = dinv
    mm = jnp.dot(h_ref[...], w_ref[...], preferred_element_type=jnp.float32)
    u_ref[...] = mm * dinv


def _tc_first(h0, W1, dA, dB):
    grid = NPAD // _BM
    return pl.pallas_call(
        _tc_first_body,
        grid=(grid,),
        in_specs=[
            pl.BlockSpec((_BM, D), lambda i: (i, 0)),
            pl.BlockSpec((D, D), lambda i: (0, 0)),
            pl.BlockSpec((_BM, 1), lambda i: (i, 0)),
            pl.BlockSpec((_BM, 1), lambda i: (i, 0)),
        ],
        out_specs=[
            pl.BlockSpec((_BM, D), lambda i: (i, 0)),
            pl.BlockSpec((_BM, 1), lambda i: (i, 0)),
        ],
        out_shape=[
            jax.ShapeDtypeStruct((NPAD, D), jnp.float32),
            jax.ShapeDtypeStruct((NPAD, 1), jnp.float32),
        ],
    )(h0, W1, dA, dB)


def _tc_mid_body(a0_ref, a1_ref, up_ref, dinv_ref, b_ref, w_ref, u_ref):
    dinv = dinv_ref[...]
    h = (a0_ref[...] + a1_ref[...] + up_ref[...]) * dinv + b_ref[...]
    h = jnp.maximum(h, 0.0)
    u_ref[...] = jnp.dot(h, w_ref[...], preferred_element_type=jnp.float32) * dinv


def _tc_mid(a0, a1, u_prev, dinv, b, W):
    grid = NPAD // _BM
    return pl.pallas_call(
        _tc_mid_body,
        grid=(grid,),
        in_specs=[
            pl.BlockSpec((_BM, D), lambda i: (i, 0)),
            pl.BlockSpec((_BM, D), lambda i: (i, 0)),
            pl.BlockSpec((_BM, D), lambda i: (i, 0)),
            pl.BlockSpec((_BM, 1), lambda i: (i, 0)),
            pl.BlockSpec((1, D), lambda i: (0, 0)),
            pl.BlockSpec((D, D), lambda i: (0, 0)),
        ],
        out_specs=pl.BlockSpec((_BM, D), lambda i: (i, 0)),
        out_shape=jax.ShapeDtypeStruct((NPAD, D), jnp.float32),
    )(a0, a1, u_prev, dinv, b, W)


_BME = 1024


def _tc_pool_body(a0_ref, a1_ref, up_ref, dinv_ref, b_ref, batch_ref,
                  fcw_ref, fcb_ref, out_ref, sums_ref, counts_ref):
    i = pl.program_id(0)
    h = (a0_ref[...] + a1_ref[...] + up_ref[...]) * dinv_ref[...] + b_ref[...]
    h = jnp.maximum(h, 0.0)
    gids = lax.broadcasted_iota(jnp.int32, (NUM_GRAPHS, _BME), 0)
    sel = (gids == batch_ref[...]).astype(jnp.float32)

    @pl.when(i == 0)
    def _():
        sums_ref[...] = jnp.zeros_like(sums_ref)
        counts_ref[...] = jnp.zeros_like(counts_ref)

    sums_ref[...] += jnp.dot(sel, h, preferred_element_type=jnp.float32,
                             precision=lax.Precision.HIGHEST)
    counts_ref[...] += jnp.sum(sel, axis=1, keepdims=True)

    @pl.when(i == pl.num_programs(0) - 1)
    def _():
        pooled = sums_ref[...] / jnp.maximum(counts_ref[...], 1.0)
        out_ref[...] = (
            jnp.dot(pooled, fcw_ref[...], preferred_element_type=jnp.float32)
            + fcb_ref[...])


def _tc_pool(a0, a1, u_prev, dinv, b, batch2d, fc_W, fc_b2d):
    grid = NPAD // _BME
    return pl.pallas_call(
        _tc_pool_body,
        grid=(grid,),
        in_specs=[
            pl.BlockSpec((_BME, D), lambda i: (i, 0)),
            pl.BlockSpec((_BME, D), lambda i: (i, 0)),
            pl.BlockSpec((_BME, D), lambda i: (i, 0)),
            pl.BlockSpec((_BME, 1), lambda i: (i, 0)),
            pl.BlockSpec((1, D), lambda i: (0, 0)),
            pl.BlockSpec((1, _BME), lambda i: (0, i)),
            pl.BlockSpec((D, 1), lambda i: (0, 0)),
            pl.BlockSpec((1, 1), lambda i: (0, 0)),
        ],
        out_specs=pl.BlockSpec((NUM_GRAPHS, 1), lambda i: (0, 0)),
        out_shape=jax.ShapeDtypeStruct((NUM_GRAPHS, 1), jnp.float32),
        scratch_shapes=[
            pltpu.VMEM((NUM_GRAPHS, D), jnp.float32),
            pltpu.VMEM((NUM_GRAPHS, 1), jnp.float32),
        ],
    )(a0, a1, u_prev, dinv, b, batch2d, fc_W, fc_b2d)


def kernel(x, edge_index, batch, emb_table, W1, b1, W2, b2, fc_W, fc_b):
    x_pad = jnp.concatenate([x, jnp.zeros((NPAD - N_NODES,), jnp.int32)])
    x_r = x_pad.reshape(NW, XNCH, CH)
    dst_r = edge_index[1].reshape(NW, ENCH, CH)
    npad_e = EPAD_W - ECH_W
    src_p = jnp.concatenate(
        [edge_index[0].reshape(NW, ECH_W),
         jnp.zeros((NW, npad_e), jnp.int32)], axis=1).reshape(NW, PNCH, PCH)
    dummy_dst = jnp.broadcast_to(
        N_NODES + jnp.arange(npad_e, dtype=jnp.int32) % (NPAD - N_NODES),
        (NW, npad_e))
    dst_p = jnp.concatenate(
        [edge_index[1].reshape(NW, ECH_W), dummy_dst],
        axis=1).reshape(NW, PNCH, PCH)
    batch_pad = jnp.concatenate(
        [batch, jnp.full((NPAD - N_NODES,), NUM_GRAPHS, jnp.int32)])
    batch2d = batch_pad.reshape(1, NPAD)

    ones_ch = jnp.ones((CH,), jnp.float32)
    zeros_rs = jnp.zeros((ROWS_S,), jnp.float32)
    zeros_ch_d = jnp.zeros((PCH, D), jnp.float32)

    sc_gather_deg, sc_propagate = _sc_kernels()
    h0, deg = sc_gather_deg(emb_table, x_r, dst_r, ones_ch, zeros_rs)
    dA = deg[0].reshape(NPAD, 1)
    dB = deg[1].reshape(NPAD, 1)

    u1, dinv = _tc_first(h0, W1, dA, dB)

    acc1 = sc_propagate(u1, src_p, dst_p, zeros_ch_d)
    a10 = acc1[0].reshape(NPAD, D)
    a11 = acc1[1].reshape(NPAD, D)

    u2 = _tc_mid(a10, a11, u1, dinv, b1.reshape(1, D), W2)

    acc2 = sc_propagate(u2, src_p, dst_p, zeros_ch_d)
    a20 = acc2[0].reshape(NPAD, D)
    a21 = acc2[1].reshape(NPAD, D)

    out = _tc_pool(a20, a21, u2, dinv, b2.reshape(1, D), batch2d,
                   fc_W, fc_b.reshape(1, 1))
    return out.reshape(NUM_GRAPHS)


# speedup vs baseline: 9.6766x; 1.0070x over previous
import functools

import jax
import jax.numpy as jnp
from jax import lax
from jax.experimental import pallas as pl
from jax.experimental.pallas import tpu as pltpu
from jax.experimental.pallas import tpu_sc as plsc

N_NODES = 10000
N_EDGES = 320000
NUM_EMB = 100000
D = 128
NUM_GRAPHS = 128

NC = 2
NS = 16
NW = NC * NS

NPAD = 10240
ROWS_W = NPAD // NW
ROWS_S = NPAD // NS

CH = 80
ECH_W = N_EDGES // NW
ENCH = ECH_W // CH
XNCH = ROWS_W // CH

PCH = 128
BCH = 4
NBLK = 20
PNCH = BCH * NBLK
EPAD_W = PNCH * PCH

@functools.lru_cache(maxsize=None)
def _sc_kernels():
    mesh = plsc.VectorSubcoreMesh(core_axis_name="c", subcore_axis_name="s",
                                  num_cores=NC, num_subcores=NS)

    @functools.partial(
        pl.kernel,
        out_type=(
            jax.ShapeDtypeStruct((NPAD, D), jnp.float32),
            jax.ShapeDtypeStruct((NC, NS, ROWS_S), jnp.float32),
        ),
        mesh=mesh,
        scratch_types=(
            pltpu.VMEM((XNCH, CH), jnp.int32),
            pltpu.VMEM((ENCH, CH), jnp.int32),
            pltpu.VMEM((2, CH, D), jnp.float32),
            pltpu.VMEM((CH,), jnp.float32),
            pltpu.VMEM((ROWS_S,), jnp.float32),
            pltpu.VMEM_SHARED((NPAD,), jnp.float32),
            pltpu.SemaphoreType.DMA,
            pltpu.SemaphoreType.DMA,
        ),
    )
    def sc_gather_deg(emb_hbm, x_hbm, dst_hbm, ones_hbm, zeros_hbm,
                      h0_hbm, deg_hbm,
                      xi_v, dst_v, rows_v, ones_v, stage_v, deg_sh, sem, dsem):
        c = lax.axis_index("c")
        s = lax.axis_index("s")
        w = c * NS + s

        pltpu.sync_copy(ones_hbm, ones_v)
        pltpu.sync_copy(zeros_hbm, stage_v)
        pltpu.sync_copy(stage_v, deg_sh.at[pl.ds(s * ROWS_S, ROWS_S)])
        pltpu.sync_copy(dst_hbm.at[w], dst_v)
        plsc.subcore_barrier()

        @pl.loop(0, ENCH)
        def _(j):
            pltpu.async_copy(ones_v, deg_sh.at[dst_v.at[j]], dsem, add=True)

        pltpu.sync_copy(x_hbm.at[w], xi_v)
        pltpu.async_copy(emb_hbm.at[xi_v.at[0]], rows_v.at[0], sem)
        for j in range(XNCH):
            pltpu.make_async_copy(emb_hbm.at[pl.ds(0, CH)], rows_v.at[j % 2],
                                  sem).wait()
            if j + 1 < XNCH:
                pltpu.async_copy(emb_hbm.at[xi_v.at[j + 1]],
                                 rows_v.at[(j + 1) % 2], sem)
            pltpu.sync_copy(rows_v.at[j % 2],
                            h0_hbm.at[pl.ds(w * ROWS_W + j * CH, CH)])

        @pl.loop(0, ENCH)
        def _(j):
            pltpu.make_async_copy(ones_v, deg_sh.at[pl.ds(0, CH)], dsem).wait()

        plsc.subcore_barrier()

        pltpu.sync_copy(deg_sh.at[pl.ds(s * ROWS_S, ROWS_S)], stage_v)
        pltpu.sync_copy(stage_v, deg_hbm.at[c].at[s])

    @functools.partial(
        pl.kernel,
        out_type=jax.ShapeDtypeStruct((NC, NS, ROWS_S, D), jnp.float32),
        mesh=mesh,
        scratch_types=(
            pltpu.VMEM((3, BCH, PCH), jnp.int32),
            pltpu.VMEM((3, BCH, PCH), jnp.int32),
            pltpu.VMEM((2, PCH, D), jnp.float32),
            pltpu.VMEM_SHARED((NPAD, D), jnp.float32),
            pltpu.SemaphoreType.DMA,
            pltpu.SemaphoreType.DMA,
            pltpu.SemaphoreType.DMA,
        ),
    )
    def sc_propagate(u_hbm, src_hbm, dst_hbm, zeros_hbm, acc_hbm,
                     src_v, dst_v, rows_v, acc_sh, gsem, ssem, isem):
        c = lax.axis_index("c")
        s = lax.axis_index("s")
        w = c * NS + s
        nrb = ROWS_S // PCH

        def fire_gather(q, j, p):
            pltpu.async_copy(u_hbm.at[src_v.at[q].at[j]], rows_v.at[p], gsem)

        def wait_gather(p):
            pltpu.make_async_copy(u_hbm.at[pl.ds(0, PCH)], rows_v.at[p],
                                  gsem).wait()

        def fire_scatter(q, j, p):
            pltpu.async_copy(rows_v.at[p], acc_sh.at[dst_v.at[q].at[j]],
                             ssem, add=True)

        def wait_ssem():
            pltpu.make_async_copy(rows_v.at[0], acc_sh.at[pl.ds(0, PCH)],
                                  ssem).wait()

        def fire_idx(b):
            pltpu.async_copy(src_hbm.at[w].at[pl.ds(b * BCH, BCH)],
                             src_v.at[b % 3], isem)
            pltpu.async_copy(dst_hbm.at[w].at[pl.ds(b * BCH, BCH)],
                             dst_v.at[b % 3], isem)

        def wait_idx():
            for ref in (src_v, dst_v):
                pltpu.make_async_copy(src_hbm.at[0].at[pl.ds(0, BCH)],
                                      ref.at[0], isem).wait()

        pltpu.sync_copy(zeros_hbm, rows_v.at[0])
        for k in range(nrb):
            pltpu.async_copy(rows_v.at[0],
                             acc_sh.at[pl.ds(s * ROWS_S + k * PCH, PCH)], ssem)
        for k in range(nrb):
            wait_ssem()
        plsc.subcore_barrier()

        fire_idx(0)
        wait_idx()
        if NBLK > 1:
            fire_idx(1)
        fire_gather(0, 0, 0)
        wait_gather(0)
        fire_gather(0, 1, 1)
        fire_scatter(0, 0, 0)

        for b in range(NBLK):
            q = b % 3

            @pl.loop(1 if b == 0 else 0, BCH - 1)
            def _(j):
                g = b * BCH + j
                p = lax.rem(g, 2)
                wait_gather(p)
                wait_ssem()
                fire_gather(q, j + 1, 1 - p)
                fire_scatter(q, j, p)

            wait_gather(1)
            wait_ssem()
            if b + 1 < NBLK:
                wait_idx()
                if b + 2 < NBLK:
                    fire_idx(b + 2)
                fire_gather((b + 1) % 3, 0, 0)
            fire_scatter(q, BCH - 1, 1)

        wait_ssem()
        plsc.subcore_barrier()

        def fire_rd(k, p):
            pltpu.async_copy(acc_sh.at[pl.ds(s * ROWS_S + k * PCH, PCH)],
                             rows_v.at[p], gsem)

        def fire_wr(k, p):
            pltpu.async_copy(rows_v.at[p],
                             acc_hbm.at[c].at[s].at[pl.ds(k * PCH, PCH)], ssem)

        def wait_rd(p):
            pltpu.make_async_copy(u_hbm.at[pl.ds(0, PCH)], rows_v.at[p],
                                  gsem).wait()

        def wait_wr():
            pltpu.make_async_copy(rows_v.at[0],
                                  acc_hbm.at[0].at[0].at[pl.ds(0, PCH)],
                                  ssem).wait()

        fire_rd(0, 0)
        for k in range(nrb):
            p = k % 2
            wait_rd(p)
            if k >= 1:
                wait_wr()
            if k + 1 < nrb:
                fire_rd(k + 1, 1 - p)
            fire_wr(k, p)
        wait_wr()

    return sc_gather_deg, sc_propagate


_BM = 2048


def _tc_first_body(h_ref, w_ref, da_ref, db_ref, u_ref, dinv_ref):
    dinv = lax.rsqrt(da_ref[...] + db_ref[...] + 1.0)
    dinv_ref[...] = dinv
    mm = jnp.dot(h_ref[...], w_ref[...], preferred_element_type=jnp.float32)
    u_ref[...] = mm * dinv


def _tc_first(h0, W1, dA, dB):
    grid = NPAD // _BM
    return pl.pallas_call(
        _tc_first_body,
        grid=(grid,),
        in_specs=[
            pl.BlockSpec((_BM, D), lambda i: (i, 0)),
            pl.BlockSpec((D, D), lambda i: (0, 0)),
            pl.BlockSpec((_BM, 1), lambda i: (i, 0)),
            pl.BlockSpec((_BM, 1), lambda i: (i, 0)),
        ],
        out_specs=[
            pl.BlockSpec((_BM, D), lambda i: (i, 0)),
            pl.BlockSpec((_BM, 1), lambda i: (i, 0)),
        ],
        out_shape=[
            jax.ShapeDtypeStruct((NPAD, D), jnp.float32),
            jax.ShapeDtypeStruct((NPAD, 1), jnp.float32),
        ],
    )(h0, W1, dA, dB)


def _tc_mid_body(a0_ref, a1_ref, up_ref, dinv_ref, b_ref, w_ref, u_ref):
    dinv = dinv_ref[...]
    h = (a0_ref[...] + a1_ref[...] + up_ref[...]) * dinv + b_ref[...]
    h = jnp.maximum(h, 0.0)
    u_ref[...] = jnp.dot(h, w_ref[...], preferred_element_type=jnp.float32) * dinv


def _tc_mid(a0, a1, u_prev, dinv, b, W):
    grid = NPAD // _BM
    return pl.pallas_call(
        _tc_mid_body,
        grid=(grid,),
        in_specs=[
            pl.BlockSpec((_BM, D), lambda i: (i, 0)),
            pl.BlockSpec((_BM, D), lambda i: (i, 0)),
            pl.BlockSpec((_BM, D), lambda i: (i, 0)),
            pl.BlockSpec((_BM, 1), lambda i: (i, 0)),
            pl.BlockSpec((1, D), lambda i: (0, 0)),
            pl.BlockSpec((D, D), lambda i: (0, 0)),
        ],
        out_specs=pl.BlockSpec((_BM, D), lambda i: (i, 0)),
        out_shape=jax.ShapeDtypeStruct((NPAD, D), jnp.float32),
    )(a0, a1, u_prev, dinv, b, W)


_BME = 1024


def _tc_pool_body(a0_ref, a1_ref, up_ref, dinv_ref, b_ref, batch_ref,
                  fcw_ref, fcb_ref, out_ref, sums_ref, counts_ref):
    i = pl.program_id(0)
    h = (a0_ref[...] + a1_ref[...] + up_ref[...]) * dinv_ref[...] + b_ref[...]
    h = jnp.maximum(h, 0.0)
    gids = lax.broadcasted_iota(jnp.int32, (NUM_GRAPHS, _BME), 0)
    sel = (gids == batch_ref[...]).astype(jnp.float32)

    @pl.when(i == 0)
    def _():
        sums_ref[...] = jnp.zeros_like(sums_ref)
        counts_ref[...] = jnp.zeros_like(counts_ref)

    sums_ref[...] += jnp.dot(sel, h, preferred_element_type=jnp.float32,
                             precision=lax.Precision.HIGHEST)
    counts_ref[...] += jnp.sum(sel, axis=1, keepdims=True)

    @pl.when(i == pl.num_programs(0) - 1)
    def _():
        pooled = sums_ref[...] / jnp.maximum(counts_ref[...], 1.0)
        out_ref[...] = (
            jnp.dot(pooled, fcw_ref[...], preferred_element_type=jnp.float32)
            + fcb_ref[...])


def _tc_pool(a0, a1, u_prev, dinv, b, batch2d, fc_W, fc_b2d):
    grid = NPAD // _BME
    return pl.pallas_call(
        _tc_pool_body,
        grid=(grid,),
        in_specs=[
            pl.BlockSpec((_BME, D), lambda i: (i, 0)),
            pl.BlockSpec((_BME, D), lambda i: (i, 0)),
            pl.BlockSpec((_BME, D), lambda i: (i, 0)),
            pl.BlockSpec((_BME, 1), lambda i: (i, 0)),
            pl.BlockSpec((1, D), lambda i: (0, 0)),
            pl.BlockSpec((1, _BME), lambda i: (0, i)),
            pl.BlockSpec((D, 1), lambda i: (0, 0)),
            pl.BlockSpec((1, 1), lambda i: (0, 0)),
        ],
        out_specs=pl.BlockSpec((NUM_GRAPHS, 1), lambda i: (0, 0)),
        out_shape=jax.ShapeDtypeStruct((NUM_GRAPHS, 1), jnp.float32),
        scratch_shapes=[
            pltpu.VMEM((NUM_GRAPHS, D), jnp.float32),
            pltpu.VMEM((NUM_GRAPHS, 1), jnp.float32),
        ],
    )(a0, a1, u_prev, dinv, b, batch2d, fc_W, fc_b2d)


def kernel(x, edge_index, batch, emb_table, W1, b1, W2, b2, fc_W, fc_b):
    x_pad = jnp.concatenate([x, jnp.zeros((NPAD - N_NODES,), jnp.int32)])
    x_r = x_pad.reshape(NW, XNCH, CH)
    dst_r = edge_index[1].reshape(NW, ENCH, CH)
    npad_e = EPAD_W - ECH_W
    src_p = jnp.concatenate(
        [edge_index[0].reshape(NW, ECH_W),
         jnp.zeros((NW, npad_e), jnp.int32)], axis=1).reshape(NW, PNCH, PCH)
    dummy_dst = jnp.broadcast_to(
        N_NODES + jnp.arange(npad_e, dtype=jnp.int32) % (NPAD - N_NODES),
        (NW, npad_e))
    dst_p = jnp.concatenate(
        [edge_index[1].reshape(NW, ECH_W), dummy_dst],
        axis=1).reshape(NW, PNCH, PCH)
    batch_pad = jnp.concatenate(
        [batch, jnp.full((NPAD - N_NODES,), NUM_GRAPHS, jnp.int32)])
    batch2d = batch_pad.reshape(1, NPAD)

    ones_ch = jnp.ones((CH,), jnp.float32)
    zeros_rs = jnp.zeros((ROWS_S,), jnp.float32)
    zeros_ch_d = jnp.zeros((PCH, D), jnp.float32)

    sc_gather_deg, sc_propagate = _sc_kernels()
    h0, deg = sc_gather_deg(emb_table, x_r, dst_r, ones_ch, zeros_rs)
    dA = deg[0].reshape(NPAD, 1)
    dB = deg[1].reshape(NPAD, 1)

    u1, dinv = _tc_first(h0, W1, dA, dB)

    acc1 = sc_propagate(u1, src_p, dst_p, zeros_ch_d)
    a10 = acc1[0].reshape(NPAD, D)
    a11 = acc1[1].reshape(NPAD, D)

    u2 = _tc_mid(a10, a11, u1, dinv, b1.reshape(1, D), W2)

    acc2 = sc_propagate(u2, src_p, dst_p, zeros_ch_d)
    a20 = acc2[0].reshape(NPAD, D)
    a21 = acc2[1].reshape(NPAD, D)

    out = _tc_pool(a20, a21, u2, dinv, b2.reshape(1, D), batch2d,
                   fc_W, fc_b.reshape(1, 1))
    return out.reshape(NUM_GRAPHS)
